# Initial kernel scaffold; baseline (speedup 1.0000x reference)
import jax
import jax.numpy as jnp
from jax.experimental import pallas as pl


def kernel(x, y, w_gate_local, w_gate_global, lW1, lb1, lW2, lb2, gW1, gb1, gW2, gb2):
    raise NotImplementedError("write your pallas kernel here")


# speedup vs baseline: 1.7289x; 1.7289x over previous
import functools

import jax
import jax.numpy as jnp
from jax.experimental import pallas as pl
from jax.experimental.pallas import tpu as pltpu

_B, _C, _H, _W = 4, 192, 32, 32
_T = _H * _W
_E = 8
_K = 2
_HID = _C * 4


def _top2(logits):
    iota = jax.lax.broadcasted_iota(jnp.int32, logits.shape, 1)
    m1 = jnp.max(logits, axis=1, keepdims=True)
    i1 = jnp.min(jnp.where(logits == m1, iota, _E), axis=1, keepdims=True)
    masked = jnp.where(iota == i1, -jnp.inf, logits)
    m2 = jnp.max(masked, axis=1, keepdims=True)
    i2 = jnp.min(jnp.where(masked == m2, iota, _E), axis=1, keepdims=True)
    e = jnp.exp(m2 - m1)
    g1 = 1.0 / (1.0 + e)
    g2 = e / (1.0 + e)
    return i1, i2, g1, g2


def _route_kernel(x_ref, y_ref, wl_ref, wg_ref, idx_ref, gate_ref):
    s = (jnp.sum(x_ref[...], axis=2) + jnp.sum(y_ref[...], axis=2)) * (0.5 / _T)
    ll = jax.lax.dot_general(s, wl_ref[...], (((1,), (0,)), ((), ())),
                             preferred_element_type=jnp.float32)
    lg = jax.lax.dot_general(s, wg_ref[...], (((1,), (0,)), ((), ())),
                             preferred_element_type=jnp.float32)
    li1, li2, lg1, lg2 = _top2(ll)
    gi1, gi2, gg1, gg2 = _top2(lg)
    idx_ref[...] = jnp.concatenate([li1, li2, gi1 + _E, gi2 + _E], axis=1)
    gate_ref[...] = jnp.concatenate([lg1, lg2, gg1, gg2], axis=1)


def _expert_kernel(idx_ref, gate_ref, x_ref, y_ref, w1_ref, b1_ref, w2_ref,
                   b2_ref, o_ref):
    del idx_ref
    b = pl.program_id(0)
    j = pl.program_id(1)
    xt = x_ref[0]
    inp = jnp.where(j < _K, xt, (xt + y_ref[0]) * 0.5)
    h = jax.lax.dot_general(w1_ref[0], inp, (((0,), (0,)), ((), ())),
                            preferred_element_type=jnp.float32)
    h = jnp.maximum(h + b1_ref[0], 0.0)
    o = jax.lax.dot_general(w2_ref[0], h, (((0,), (0,)), ((), ())),
                            preferred_element_type=jnp.float32)
    o = (o + b2_ref[0]) * gate_ref[b, j]

    @pl.when(j == 0)
    def _init():
        o_ref[0] = o

    @pl.when(j != 0)
    def _acc():
        o_ref[0] += o


@functools.partial(jax.jit, static_argnames=())
def kernel(x, y, w_gate_local, w_gate_global, lW1, lb1, lW2, lb2, gW1, gb1,
           gW2, gb2):
    xr = x.reshape(_B, _C, _T)
    yr = y.reshape(_B, _C, _T)

    idx, gates = pl.pallas_call(
        _route_kernel,
        out_shape=(
            jax.ShapeDtypeStruct((_B, 2 * _K), jnp.int32),
            jax.ShapeDtypeStruct((_B, 2 * _K), jnp.float32),
        ),
    )(xr, yr, w_gate_local, w_gate_global)

    w1 = jnp.concatenate([lW1, gW1], axis=0)
    b1 = jnp.concatenate([lb1, gb1], axis=0).reshape(2 * _E, _HID, 1)
    w2 = jnp.concatenate([lW2, gW2], axis=0)
    b2 = jnp.concatenate([lb2, gb2], axis=0).reshape(2 * _E, _C, 1)

    grid = (_B, 2 * _K)
    out = pl.pallas_call(
        _expert_kernel,
        grid_spec=pltpu.PrefetchScalarGridSpec(
            num_scalar_prefetch=2,
            grid=grid,
            in_specs=[
                pl.BlockSpec((1, _C, _T), lambda b, j, idx, g: (b, 0, 0)),
                pl.BlockSpec((1, _C, _T), lambda b, j, idx, g: (b, 0, 0)),
                pl.BlockSpec((1, _C, _HID),
                             lambda b, j, idx, g: (idx[b, j], 0, 0)),
                pl.BlockSpec((1, _HID, 1),
                             lambda b, j, idx, g: (idx[b, j], 0, 0)),
                pl.BlockSpec((1, _HID, _C),
                             lambda b, j, idx, g: (idx[b, j], 0, 0)),
                pl.BlockSpec((1, _C, 1),
                             lambda b, j, idx, g: (idx[b, j], 0, 0)),
            ],
            out_specs=pl.BlockSpec((1, _C, _T), lambda b, j, idx, g: (b, 0, 0)),
        ),
        out_shape=jax.ShapeDtypeStruct((_B, _C, _T), jnp.float32),
    )(idx, gates, xr, yr, w1, b1, w2, b2)

    return out.reshape(_B, _C, _H, _W)


# speedup vs baseline: 2.2006x; 1.2729x over previous
import jax
import jax.numpy as jnp
from jax.experimental import pallas as pl
from jax.experimental.pallas import tpu as pltpu

_B, _C, _H, _W = 4, 192, 32, 32
_T = _H * _W
_E = 8
_K = 2
_HID = _C * 4


def _top2(logits):
    iota = jax.lax.broadcasted_iota(jnp.int32, logits.shape, 1)
    m1 = jnp.max(logits, axis=1, keepdims=True)
    i1 = jnp.min(jnp.where(logits == m1, iota, _E), axis=1, keepdims=True)
    masked = jnp.where(iota == i1, -jnp.inf, logits)
    m2 = jnp.max(masked, axis=1, keepdims=True)
    i2 = jnp.min(jnp.where(masked == m2, iota, _E), axis=1, keepdims=True)
    e = jnp.exp(m2 - m1)
    g1 = 1.0 / (1.0 + e)
    g2 = e / (1.0 + e)
    return i1, i2, g1, g2


def _route_kernel(x_ref, y_ref, wl_ref, wg_ref, idx_ref, gate_ref):
    s = (jnp.sum(x_ref[...], axis=2) + jnp.sum(y_ref[...], axis=2)) * (0.5 / _T)
    ll = jax.lax.dot_general(s, wl_ref[...], (((1,), (0,)), ((), ())),
                             preferred_element_type=jnp.float32)
    lg = jax.lax.dot_general(s, wg_ref[...], (((1,), (0,)), ((), ())),
                             preferred_element_type=jnp.float32)
    li1, li2, lg1, lg2 = _top2(ll)
    gi1, gi2, gg1, gg2 = _top2(lg)
    idx_ref[...] = jnp.concatenate([li1, li2, gi1, gi2], axis=1)
    gate_ref[...] = jnp.concatenate([lg1, lg2, gg1, gg2], axis=1)


def _ffn(w1, b1, w2, b2, inp):
    h = jax.lax.dot_general(w1, inp, (((0,), (0,)), ((), ())),
                            preferred_element_type=jnp.float32)
    h = jnp.maximum(h + b1, 0.0)
    return jax.lax.dot_general(w2, h, (((0,), (0,)), ((), ())),
                               preferred_element_type=jnp.float32) + b2


def _expert_kernel(idx_ref, gate_ref, x_ref, y_ref, lw1_ref, lb1_ref, lw2_ref,
                   lb2_ref, gw1_ref, gb1_ref, gw2_ref, gb2_ref, o_ref):
    del idx_ref
    b = pl.program_id(0)
    s = pl.program_id(1)
    xt = x_ref[0]
    fu = (xt + y_ref[0]) * 0.5
    ol = _ffn(lw1_ref[0], lb1_ref[0], lw2_ref[0], lb2_ref[0], xt)
    og = _ffn(gw1_ref[0], gb1_ref[0], gw2_ref[0], gb2_ref[0], fu)
    acc = gate_ref[b, s] * ol + gate_ref[b, _K + s] * og

    @pl.when(s == 0)
    def _init():
        o_ref[0] = acc

    @pl.when(s != 0)
    def _acc():
        o_ref[0] += acc


def kernel(x, y, w_gate_local, w_gate_global, lW1, lb1, lW2, lb2, gW1, gb1,
           gW2, gb2):
    xr = x.reshape(_B, _C, _T)
    yr = y.reshape(_B, _C, _T)

    idx, gates = pl.pallas_call(
        _route_kernel,
        out_shape=(
            jax.ShapeDtypeStruct((_B, 2 * _K), jnp.int32),
            jax.ShapeDtypeStruct((_B, 2 * _K), jnp.float32),
        ),
    )(xr, yr, w_gate_local, w_gate_global)

    lb1r = lb1.reshape(_E, _HID, 1)
    lb2r = lb2.reshape(_E, _C, 1)
    gb1r = gb1.reshape(_E, _HID, 1)
    gb2r = gb2.reshape(_E, _C, 1)

    grid = (_B, _K)
    out = pl.pallas_call(
        _expert_kernel,
        grid_spec=pltpu.PrefetchScalarGridSpec(
            num_scalar_prefetch=2,
            grid=grid,
            in_specs=[
                pl.BlockSpec((1, _C, _T), lambda b, s, idx, g: (b, 0, 0)),
                pl.BlockSpec((1, _C, _T), lambda b, s, idx, g: (b, 0, 0)),
                pl.BlockSpec((1, _C, _HID),
                             lambda b, s, idx, g: (idx[b, s], 0, 0)),
                pl.BlockSpec((1, _HID, 1),
                             lambda b, s, idx, g: (idx[b, s], 0, 0)),
                pl.BlockSpec((1, _HID, _C),
                             lambda b, s, idx, g: (idx[b, s], 0, 0)),
                pl.BlockSpec((1, _C, 1),
                             lambda b, s, idx, g: (idx[b, s], 0, 0)),
                pl.BlockSpec((1, _C, _HID),
                             lambda b, s, idx, g: (idx[b, _K + s], 0, 0)),
                pl.BlockSpec((1, _HID, 1),
                             lambda b, s, idx, g: (idx[b, _K + s], 0, 0)),
                pl.BlockSpec((1, _HID, _C),
                             lambda b, s, idx, g: (idx[b, _K + s], 0, 0)),
                pl.BlockSpec((1, _C, 1),
                             lambda b, s, idx, g: (idx[b, _K + s], 0, 0)),
            ],
            out_specs=pl.BlockSpec((1, _C, _T), lambda b, s, idx, g: (b, 0, 0)),
        ),
        out_shape=jax.ShapeDtypeStruct((_B, _C, _T), jnp.float32),
    )(idx, gates, xr, yr, lW1, lb1r, lW2, lb2r, gW1, gb1r, gW2, gb2r)

    return out.reshape(_B, _C, _H, _W)


# speedup vs baseline: 2.2224x; 1.0099x over previous
import jax
import jax.numpy as jnp
from jax.experimental import pallas as pl
from jax.experimental.pallas import tpu as pltpu

_B, _C, _H, _W = 4, 192, 32, 32
_T = _H * _W
_E = 8
_K = 2
_HID = _C * 4


def _top2(logits):
    iota = jax.lax.broadcasted_iota(jnp.int32, logits.shape, 1)
    m1 = jnp.max(logits, axis=1, keepdims=True)
    i1 = jnp.min(jnp.where(logits == m1, iota, _E), axis=1, keepdims=True)
    masked = jnp.where(iota == i1, -jnp.inf, logits)
    m2 = jnp.max(masked, axis=1, keepdims=True)
    i2 = jnp.min(jnp.where(masked == m2, iota, _E), axis=1, keepdims=True)
    e = jnp.exp(m2 - m1)
    g1 = 1.0 / (1.0 + e)
    g2 = e / (1.0 + e)
    return i1, i2, g1, g2


def _route_kernel(x_ref, y_ref, wl_ref, wg_ref, idx_ref, gate_ref):
    s = (jnp.sum(x_ref[...], axis=2) + jnp.sum(y_ref[...], axis=2)) * (0.5 / _T)
    ll = jax.lax.dot_general(s, wl_ref[...], (((1,), (0,)), ((), ())),
                             preferred_element_type=jnp.float32)
    lg = jax.lax.dot_general(s, wg_ref[...], (((1,), (0,)), ((), ())),
                             preferred_element_type=jnp.float32)
    li1, li2, lg1, lg2 = _top2(ll)
    gi1, gi2, gg1, gg2 = _top2(lg)
    idx_ref[...] = jnp.concatenate([li1, li2, gi1, gi2], axis=1)
    gate_ref[...] = jnp.concatenate([lg1, lg2, gg1, gg2], axis=1)


def _ffn(w1, b1, w2, b2, inp):
    h = jax.lax.dot_general(w1.astype(jnp.bfloat16), inp,
                            (((0,), (0,)), ((), ())),
                            preferred_element_type=jnp.float32)
    h = jnp.maximum(h + b1, 0.0).astype(jnp.bfloat16)
    return jax.lax.dot_general(w2.astype(jnp.bfloat16), h,
                               (((0,), (0,)), ((), ())),
                               preferred_element_type=jnp.float32) + b2


def _expert_kernel(idx_ref, gate_ref, x_ref, y_ref, lw1_ref, lb1_ref, lw2_ref,
                   lb2_ref, gw1_ref, gb1_ref, gw2_ref, gb2_ref, o_ref):
    del idx_ref
    b = pl.program_id(0)
    s = pl.program_id(1)
    xt = x_ref[0]
    fu = ((xt + y_ref[0]) * 0.5).astype(jnp.bfloat16)
    ol = _ffn(lw1_ref[0], lb1_ref[0], lw2_ref[0], lb2_ref[0],
              xt.astype(jnp.bfloat16))
    og = _ffn(gw1_ref[0], gb1_ref[0], gw2_ref[0], gb2_ref[0], fu)
    acc = gate_ref[b, s] * ol + gate_ref[b, _K + s] * og

    @pl.when(s == 0)
    def _init():
        o_ref[0] = acc

    @pl.when(s != 0)
    def _acc():
        o_ref[0] += acc


def kernel(x, y, w_gate_local, w_gate_global, lW1, lb1, lW2, lb2, gW1, gb1,
           gW2, gb2):
    xr = x.reshape(_B, _C, _T)
    yr = y.reshape(_B, _C, _T)

    idx, gates = pl.pallas_call(
        _route_kernel,
        out_shape=(
            jax.ShapeDtypeStruct((_B, 2 * _K), jnp.int32),
            jax.ShapeDtypeStruct((_B, 2 * _K), jnp.float32),
        ),
    )(xr, yr, w_gate_local, w_gate_global)

    lb1r = lb1.reshape(_E, _HID, 1)
    lb2r = lb2.reshape(_E, _C, 1)
    gb1r = gb1.reshape(_E, _HID, 1)
    gb2r = gb2.reshape(_E, _C, 1)

    grid = (_B, _K)
    out = pl.pallas_call(
        _expert_kernel,
        grid_spec=pltpu.PrefetchScalarGridSpec(
            num_scalar_prefetch=2,
            grid=grid,
            in_specs=[
                pl.BlockSpec((1, _C, _T), lambda b, s, idx, g: (b, 0, 0)),
                pl.BlockSpec((1, _C, _T), lambda b, s, idx, g: (b, 0, 0)),
                pl.BlockSpec((1, _C, _HID),
                             lambda b, s, idx, g: (idx[b, s], 0, 0)),
                pl.BlockSpec((1, _HID, 1),
                             lambda b, s, idx, g: (idx[b, s], 0, 0)),
                pl.BlockSpec((1, _HID, _C),
                             lambda b, s, idx, g: (idx[b, s], 0, 0)),
                pl.BlockSpec((1, _C, 1),
                             lambda b, s, idx, g: (idx[b, s], 0, 0)),
                pl.BlockSpec((1, _C, _HID),
                             lambda b, s, idx, g: (idx[b, _K + s], 0, 0)),
                pl.BlockSpec((1, _HID, 1),
                             lambda b, s, idx, g: (idx[b, _K + s], 0, 0)),
                pl.BlockSpec((1, _HID, _C),
                             lambda b, s, idx, g: (idx[b, _K + s], 0, 0)),
                pl.BlockSpec((1, _C, 1),
                             lambda b, s, idx, g: (idx[b, _K + s], 0, 0)),
            ],
            out_specs=pl.BlockSpec((1, _C, _T), lambda b, s, idx, g: (b, 0, 0)),
        ),
        out_shape=jax.ShapeDtypeStruct((_B, _C, _T), jnp.float32),
    )(idx, gates, xr, yr, lW1, lb1r, lW2, lb2r, gW1, gb1r, gW2, gb2r)

    return out.reshape(_B, _C, _H, _W)
